# baseline (speedup 1.0000x reference)
import functools

import jax
import jax.numpy as jnp
from jax import lax
from jax.experimental import pallas as pl
from jax.experimental.pallas import tpu as pltpu
from jax.experimental.pallas import tpu_sc as plsc

B = 128
K = 32768
D = 1024
TOPK = 8
KB = 2048
NUM_KB = K // KB

_NEG_INF = float("-inf")

_SORT8 = [(0, 1), (2, 3), (4, 5), (6, 7), (0, 2), (1, 3), (4, 6), (5, 7),
          (1, 2), (5, 6), (0, 4), (1, 5), (2, 6), (3, 7), (2, 4), (3, 5),
          (1, 2), (3, 4), (5, 6)]
_BITONIC8 = [(0, 4), (1, 5), (2, 6), (3, 7), (0, 2), (1, 3), (4, 6), (5, 7),
             (0, 1), (2, 3), (4, 5), (6, 7)]


def _ce(va, ia, vb, ib):
    m = va >= vb
    return (jnp.where(m, va, vb), jnp.where(m, ia, ib),
            jnp.where(m, vb, va), jnp.where(m, ib, ia))


def _apply_net(net, vs, js):
    vs = list(vs)
    js = list(js)
    for a, b in net:
        vs[a], js[a], vs[b], js[b] = _ce(vs[a], js[a], vs[b], js[b])
    return vs, js


def _top8_of_union(rv, ri, dv, di):
    cs_v = []
    cs_i = []
    for i in range(8):
        m = rv[i] >= dv[7 - i]
        cs_v.append(jnp.where(m, rv[i], dv[7 - i]))
        cs_i.append(jnp.where(m, ri[i], di[7 - i]))
    return _apply_net(_BITONIC8, cs_v, cs_i)


def _qproj_body(qh_ref, wq_ref, bq_ref, qnt_ref):
    qp_t = lax.dot_general(
        wq_ref[...].astype(jnp.bfloat16), qh_ref[...].astype(jnp.bfloat16),
        dimension_numbers=(((1,), (1,)), ((), ())),
        preferred_element_type=jnp.float32)
    qp_t = qp_t + bq_ref[...]
    n2 = jnp.sum(qp_t * qp_t, axis=0, keepdims=True)
    qnt_ref[...] = qp_t / jnp.maximum(jnp.sqrt(n2), 1e-12)


def _qproj(query_hidden, W_q, b_q):
    return pl.pallas_call(
        _qproj_body,
        out_shape=jax.ShapeDtypeStruct((D, B), jnp.float32),
    )(query_hidden, W_q, b_q.reshape(D, 1))


def _merge_block(st, idx, cv_ref, ci_ref):
    g = KB // 64
    sv = st.reshape(g, 8, 8, B)
    si = idx.reshape(g, 8, 8, B)
    vs = [sv[:, c] for c in range(8)]
    js = [si[:, c] for c in range(8)]
    vs, js = _apply_net(_SORT8, vs, js)
    while g > 1:
        h = g // 2
        vs, js = _top8_of_union([v[:h] for v in vs], [x[:h] for x in js],
                                [v[h:] for v in vs], [x[h:] for x in js])
        g = h
    vs = [v[0] for v in vs]
    js = [x[0] for x in js]
    cv = cv_ref[...]
    ci = ci_ref[...]
    rv = [cv[8 * t:8 * t + 8, :] for t in range(8)]
    ri = [ci[8 * t:8 * t + 8, :] for t in range(8)]
    nv, ni = _top8_of_union(rv, ri, vs, js)
    cv_ref[...] = jnp.concatenate(nv, axis=0)
    ci_ref[...] = jnp.concatenate(ni, axis=0)


def _process_block(kbuf_ref, qnt, jnum, cv_ref, ci_ref):
    k = kbuf_ref[...]
    n2 = jnp.sum(k * k, axis=1, keepdims=True)
    kn = k / jnp.maximum(jnp.sqrt(n2), 1e-12)
    st = lax.dot_general(
        kn.astype(jnp.bfloat16), qnt,
        dimension_numbers=(((1,), (0,)), ((), ())),
        preferred_element_type=jnp.float32)
    idx = lax.broadcasted_iota(jnp.int32, (KB, B), 0) + jnum * KB
    _merge_block(st, idx, cv_ref, ci_ref)


def _simstopk_body(qh_ref, wq_ref, bq_ref, keys_hbm, tv_ref, ti_ref,
                   kb0, kb1, qnt_ref, cv_ref, ci_ref, sem0, sem1):
    pltpu.make_async_copy(keys_hbm.at[pl.ds(0, KB), :], kb0, sem0).start()
    pltpu.make_async_copy(keys_hbm.at[pl.ds(KB, KB), :], kb1, sem1).start()

    qp_t = lax.dot_general(
        wq_ref[...].astype(jnp.bfloat16), qh_ref[...].astype(jnp.bfloat16),
        dimension_numbers=(((1,), (1,)), ((), ())),
        preferred_element_type=jnp.float32)
    qp_t = qp_t + bq_ref[...]
    n2 = jnp.sum(qp_t * qp_t, axis=0, keepdims=True)
    qnt_ref[...] = qp_t / jnp.maximum(jnp.sqrt(n2), 1e-12)

    cv_ref[...] = jnp.full((64, B), _NEG_INF, jnp.float32)
    ci_ref[...] = jnp.zeros((64, B), jnp.int32)

    def pair(i, _):
        j0 = 2 * i
        qnt = qnt_ref[...].astype(jnp.bfloat16)
        pltpu.make_async_copy(keys_hbm.at[pl.ds(j0 * KB, KB), :],
                              kb0, sem0).wait()
        _process_block(kb0, qnt, j0, cv_ref, ci_ref)

        @pl.when(j0 + 2 < NUM_KB)
        def _():
            pltpu.make_async_copy(keys_hbm.at[pl.ds((j0 + 2) * KB, KB), :],
                                  kb0, sem0).start()

        pltpu.make_async_copy(keys_hbm.at[pl.ds((j0 + 1) * KB, KB), :],
                              kb1, sem1).wait()
        _process_block(kb1, qnt, j0 + 1, cv_ref, ci_ref)

        @pl.when(j0 + 3 < NUM_KB)
        def _():
            pltpu.make_async_copy(keys_hbm.at[pl.ds((j0 + 3) * KB, KB), :],
                                  kb1, sem1).start()

        return _

    lax.fori_loop(0, NUM_KB // 2, pair, None)

    cand_v = cv_ref[...]
    cand_i = ci_ref[...]
    rows_v = []
    rows_i = []
    for _ in range(TOPK):
        m = jnp.max(cand_v, axis=0, keepdims=True)
        eq = cand_v == m
        pick = jnp.min(jnp.where(eq, cand_i, jnp.int32(2**31 - 1)),
                       axis=0, keepdims=True)
        rows_v.append(m)
        rows_i.append(pick)
        cand_v = jnp.where(eq, _NEG_INF, cand_v)
    tv_ref[...] = jnp.concatenate(rows_v, axis=0)
    ti_ref[...] = jnp.concatenate(rows_i, axis=0)


def _simstopk(query_hidden, W_q, b_q, keys):
    return pl.pallas_call(
        _simstopk_body,
        in_specs=[
            pl.BlockSpec(memory_space=pltpu.VMEM),
            pl.BlockSpec(memory_space=pltpu.VMEM),
            pl.BlockSpec(memory_space=pltpu.VMEM),
            pl.BlockSpec(memory_space=pl.ANY),
        ],
        out_specs=[
            pl.BlockSpec(memory_space=pltpu.VMEM),
            pl.BlockSpec(memory_space=pltpu.VMEM),
        ],
        out_shape=[
            jax.ShapeDtypeStruct((TOPK, B), jnp.float32),
            jax.ShapeDtypeStruct((TOPK, B), jnp.int32),
        ],
        scratch_shapes=[
            pltpu.VMEM((KB, D), jnp.float32),
            pltpu.VMEM((KB, D), jnp.float32),
            pltpu.VMEM((D, B), jnp.float32),
            pltpu.VMEM((64, B), jnp.float32),
            pltpu.VMEM((64, B), jnp.int32),
            pltpu.SemaphoreType.DMA,
            pltpu.SemaphoreType.DMA,
        ],
    )(query_hidden, W_q, b_q.reshape(D, 1), keys)


_NW = 32
_ROWS_PER_W = (B * TOPK) // _NW


def _sc_gather(values, idx_flat):
    mesh = plsc.VectorSubcoreMesh(core_axis_name="c", subcore_axis_name="s")

    @functools.partial(
        pl.kernel,
        mesh=mesh,
        out_type=jax.ShapeDtypeStruct((B * TOPK, D), jnp.float32),
        scratch_types=[
            pltpu.VMEM((_ROWS_PER_W,), jnp.int32),
            pltpu.VMEM((_ROWS_PER_W, D), jnp.float32),
            pltpu.SemaphoreType.DMA,
        ],
    )
    def _gather_kernel(values_hbm, idx_hbm, out_hbm, idx_v, rows_v, sem):
        wid = lax.axis_index("s") * 2 + lax.axis_index("c")
        base = wid * _ROWS_PER_W
        pltpu.sync_copy(idx_hbm.at[pl.ds(base, _ROWS_PER_W)], idx_v)
        pltpu.async_copy(values_hbm.at[idx_v], rows_v, sem).wait()
        pltpu.sync_copy(rows_v, out_hbm.at[pl.ds(base, _ROWS_PER_W)])

    return _gather_kernel(values, idx_flat)


def _final_body(g_ref, tv_ref, tok_ref, wv_ref, bv_ref, mg_ref,
                fused_ref, nov_ref):
    tvt = jnp.transpose(tv_ref[...], (1, 0))
    m = jnp.max(tvt, axis=1, keepdims=True)
    e = jnp.exp(tvt - m)
    w = e / jnp.sum(e, axis=1, keepdims=True)
    nov_ref[...] = 1.0 - m

    r = jnp.zeros((B, D), jnp.float32)
    for i in range(TOPK):
        r = r + g_ref[pl.ds(i * B, B), :] * w[:, i:i + 1]

    mh = lax.dot_general(
        r.astype(jnp.bfloat16), wv_ref[...].astype(jnp.bfloat16),
        dimension_numbers=(((1,), (1,)), ((), ())),
        preferred_element_type=jnp.float32)
    mh = mh + bv_ref[...]
    gate = jax.nn.sigmoid(mg_ref[0, 0])
    fused_ref[...] = (1.0 - gate) * tok_ref[...] + gate * mh


def _final(gathered, top_vals, token_embed, W_v, b_v, memory_gate):
    return pl.pallas_call(
        _final_body,
        out_shape=[
            jax.ShapeDtypeStruct((B, D), jnp.float32),
            jax.ShapeDtypeStruct((B, 1), jnp.float32),
        ],
    )(gathered, top_vals, token_embed, W_v, b_v.reshape(1, D),
      memory_gate.reshape(1, 1))


def kernel(query_hidden, keys, values, token_embed, W_q, b_q, W_v, b_v,
           memory_gate):
    top_vals, top_idx = _simstopk(query_hidden, W_q, b_q, keys)
    gathered = _sc_gather(values, top_idx.reshape(B * TOPK))
    fused, novelty = _final(gathered, top_vals, token_embed, W_v, b_v,
                            memory_gate)
    return fused, novelty.reshape(B)


# speedup vs baseline: 1.0064x; 1.0064x over previous
import functools

import jax
import jax.numpy as jnp
from jax import lax
from jax.experimental import pallas as pl
from jax.experimental.pallas import tpu as pltpu
from jax.experimental.pallas import tpu_sc as plsc

B = 128
K = 32768
D = 1024
TOPK = 8
KB = 1024
NBUF = 4
NUM_KB = K // KB

_NEG_INF = float("-inf")

_SORT8 = [(0, 1), (2, 3), (4, 5), (6, 7), (0, 2), (1, 3), (4, 6), (5, 7),
          (1, 2), (5, 6), (0, 4), (1, 5), (2, 6), (3, 7), (2, 4), (3, 5),
          (1, 2), (3, 4), (5, 6)]
_BITONIC8 = [(0, 4), (1, 5), (2, 6), (3, 7), (0, 2), (1, 3), (4, 6), (5, 7),
             (0, 1), (2, 3), (4, 5), (6, 7)]


def _ce(va, ia, vb, ib):
    m = va >= vb
    return (jnp.where(m, va, vb), jnp.where(m, ia, ib),
            jnp.where(m, vb, va), jnp.where(m, ib, ia))


def _apply_net(net, vs, js):
    vs = list(vs)
    js = list(js)
    for a, b in net:
        vs[a], js[a], vs[b], js[b] = _ce(vs[a], js[a], vs[b], js[b])
    return vs, js


def _top8_of_union(rv, ri, dv, di):
    cs_v = []
    cs_i = []
    for i in range(8):
        m = rv[i] >= dv[7 - i]
        cs_v.append(jnp.where(m, rv[i], dv[7 - i]))
        cs_i.append(jnp.where(m, ri[i], di[7 - i]))
    return _apply_net(_BITONIC8, cs_v, cs_i)


def _qproj_body(qh_ref, wq_ref, bq_ref, qnt_ref):
    qp_t = lax.dot_general(
        wq_ref[...].astype(jnp.bfloat16), qh_ref[...].astype(jnp.bfloat16),
        dimension_numbers=(((1,), (1,)), ((), ())),
        preferred_element_type=jnp.float32)
    qp_t = qp_t + bq_ref[...]
    n2 = jnp.sum(qp_t * qp_t, axis=0, keepdims=True)
    qnt_ref[...] = qp_t / jnp.maximum(jnp.sqrt(n2), 1e-12)


def _qproj(query_hidden, W_q, b_q):
    return pl.pallas_call(
        _qproj_body,
        out_shape=jax.ShapeDtypeStruct((D, B), jnp.float32),
    )(query_hidden, W_q, b_q.reshape(D, 1))


def _merge_block(st, idx, cv_ref, ci_ref):
    g = KB // 64
    sv = st.reshape(g, 8, 8, B)
    si = idx.reshape(g, 8, 8, B)
    vs = [sv[:, c] for c in range(8)]
    js = [si[:, c] for c in range(8)]
    vs, js = _apply_net(_SORT8, vs, js)
    while g > 1:
        h = g // 2
        vs, js = _top8_of_union([v[:h] for v in vs], [x[:h] for x in js],
                                [v[h:] for v in vs], [x[h:] for x in js])
        g = h
    vs = [v[0] for v in vs]
    js = [x[0] for x in js]
    cv = cv_ref[...]
    ci = ci_ref[...]
    rv = [cv[8 * t:8 * t + 8, :] for t in range(8)]
    ri = [ci[8 * t:8 * t + 8, :] for t in range(8)]
    nv, ni = _top8_of_union(rv, ri, vs, js)
    cv_ref[...] = jnp.concatenate(nv, axis=0)
    ci_ref[...] = jnp.concatenate(ni, axis=0)


def _process_block(kbuf_ref, qnt, jnum, cv_ref, ci_ref):
    k = kbuf_ref[...]
    n2 = jnp.sum(k * k, axis=1, keepdims=True)
    kn = k / jnp.maximum(jnp.sqrt(n2), 1e-12)
    st = lax.dot_general(
        kn.astype(jnp.bfloat16), qnt,
        dimension_numbers=(((1,), (0,)), ((), ())),
        preferred_element_type=jnp.float32)
    idx = lax.broadcasted_iota(jnp.int32, (KB, B), 0) + jnum * KB
    _merge_block(st, idx, cv_ref, ci_ref)


def _simstopk_body(qh_ref, wq_ref, bq_ref, keys_hbm, tv_ref, ti_ref,
                   kb0, kb1, kb2, kb3, qnt_ref, cv_ref, ci_ref,
                   sem0, sem1, sem2, sem3):
    kbs = (kb0, kb1, kb2, kb3)
    sems = (sem0, sem1, sem2, sem3)
    for b in range(NBUF):
        pltpu.make_async_copy(keys_hbm.at[pl.ds(b * KB, KB), :],
                              kbs[b], sems[b]).start()

    qp_t = lax.dot_general(
        wq_ref[...].astype(jnp.bfloat16), qh_ref[...].astype(jnp.bfloat16),
        dimension_numbers=(((1,), (1,)), ((), ())),
        preferred_element_type=jnp.float32)
    qp_t = qp_t + bq_ref[...]
    n2 = jnp.sum(qp_t * qp_t, axis=0, keepdims=True)
    qnt_ref[...] = qp_t / jnp.maximum(jnp.sqrt(n2), 1e-12)

    cv_ref[...] = jnp.full((64, B), _NEG_INF, jnp.float32)
    ci_ref[...] = jnp.zeros((64, B), jnp.int32)

    def round4(i, _):
        j0 = NBUF * i
        qnt = qnt_ref[...].astype(jnp.bfloat16)
        for b in range(NBUF):
            j = j0 + b
            pltpu.make_async_copy(keys_hbm.at[pl.ds(j * KB, KB), :],
                                  kbs[b], sems[b]).wait()
            _process_block(kbs[b], qnt, j, cv_ref, ci_ref)

            @pl.when(j + NBUF < NUM_KB)
            def _():
                pltpu.make_async_copy(
                    keys_hbm.at[pl.ds((j + NBUF) * KB, KB), :],
                    kbs[b], sems[b]).start()

        return _

    lax.fori_loop(0, NUM_KB // NBUF, round4, None)

    cand_v = cv_ref[...]
    cand_i = ci_ref[...]
    rows_v = []
    rows_i = []
    for _ in range(TOPK):
        m = jnp.max(cand_v, axis=0, keepdims=True)
        eq = cand_v == m
        pick = jnp.min(jnp.where(eq, cand_i, jnp.int32(2**31 - 1)),
                       axis=0, keepdims=True)
        rows_v.append(m)
        rows_i.append(pick)
        cand_v = jnp.where(eq, _NEG_INF, cand_v)
    tv_ref[...] = jnp.concatenate(rows_v, axis=0)
    ti_ref[...] = jnp.concatenate(rows_i, axis=0)


def _simstopk(query_hidden, W_q, b_q, keys):
    return pl.pallas_call(
        _simstopk_body,
        in_specs=[
            pl.BlockSpec(memory_space=pltpu.VMEM),
            pl.BlockSpec(memory_space=pltpu.VMEM),
            pl.BlockSpec(memory_space=pltpu.VMEM),
            pl.BlockSpec(memory_space=pl.ANY),
        ],
        out_specs=[
            pl.BlockSpec(memory_space=pltpu.VMEM),
            pl.BlockSpec(memory_space=pltpu.VMEM),
        ],
        out_shape=[
            jax.ShapeDtypeStruct((TOPK, B), jnp.float32),
            jax.ShapeDtypeStruct((TOPK, B), jnp.int32),
        ],
        scratch_shapes=[
            pltpu.VMEM((KB, D), jnp.float32),
            pltpu.VMEM((KB, D), jnp.float32),
            pltpu.VMEM((KB, D), jnp.float32),
            pltpu.VMEM((KB, D), jnp.float32),
            pltpu.VMEM((D, B), jnp.float32),
            pltpu.VMEM((64, B), jnp.float32),
            pltpu.VMEM((64, B), jnp.int32),
            pltpu.SemaphoreType.DMA,
            pltpu.SemaphoreType.DMA,
            pltpu.SemaphoreType.DMA,
            pltpu.SemaphoreType.DMA,
        ],
    )(query_hidden, W_q, b_q.reshape(D, 1), keys)


_NW = 32
_ROWS_PER_W = (B * TOPK) // _NW


def _sc_gather(values, idx_flat):
    mesh = plsc.VectorSubcoreMesh(core_axis_name="c", subcore_axis_name="s")

    @functools.partial(
        pl.kernel,
        mesh=mesh,
        out_type=jax.ShapeDtypeStruct((B * TOPK, D), jnp.float32),
        scratch_types=[
            pltpu.VMEM((_ROWS_PER_W,), jnp.int32),
            pltpu.VMEM((_ROWS_PER_W, D), jnp.float32),
            pltpu.SemaphoreType.DMA,
        ],
    )
    def _gather_kernel(values_hbm, idx_hbm, out_hbm, idx_v, rows_v, sem):
        wid = lax.axis_index("s") * 2 + lax.axis_index("c")
        base = wid * _ROWS_PER_W
        pltpu.sync_copy(idx_hbm.at[pl.ds(base, _ROWS_PER_W)], idx_v)
        pltpu.async_copy(values_hbm.at[idx_v], rows_v, sem).wait()
        pltpu.sync_copy(rows_v, out_hbm.at[pl.ds(base, _ROWS_PER_W)])

    return _gather_kernel(values, idx_flat)


def _final_body(g_ref, tv_ref, tok_ref, wv_ref, bv_ref, mg_ref,
                fused_ref, nov_ref):
    tvt = jnp.transpose(tv_ref[...], (1, 0))
    m = jnp.max(tvt, axis=1, keepdims=True)
    e = jnp.exp(tvt - m)
    w = e / jnp.sum(e, axis=1, keepdims=True)
    nov_ref[...] = 1.0 - m

    r = jnp.zeros((B, D), jnp.float32)
    for i in range(TOPK):
        r = r + g_ref[pl.ds(i * B, B), :] * w[:, i:i + 1]

    mh = lax.dot_general(
        r.astype(jnp.bfloat16), wv_ref[...].astype(jnp.bfloat16),
        dimension_numbers=(((1,), (1,)), ((), ())),
        preferred_element_type=jnp.float32)
    mh = mh + bv_ref[...]
    gate = jax.nn.sigmoid(mg_ref[0, 0])
    fused_ref[...] = (1.0 - gate) * tok_ref[...] + gate * mh


def _final(gathered, top_vals, token_embed, W_v, b_v, memory_gate):
    return pl.pallas_call(
        _final_body,
        out_shape=[
            jax.ShapeDtypeStruct((B, D), jnp.float32),
            jax.ShapeDtypeStruct((B, 1), jnp.float32),
        ],
    )(gathered, top_vals, token_embed, W_v, b_v.reshape(1, D),
      memory_gate.reshape(1, 1))


def kernel(query_hidden, keys, values, token_embed, W_q, b_q, W_v, b_v,
           memory_gate):
    top_vals, top_idx = _simstopk(query_hidden, W_q, b_q, keys)
    gathered = _sc_gather(values, top_idx.reshape(B * TOPK))
    fused, novelty = _final(gathered, top_vals, token_embed, W_v, b_v,
                            memory_gate)
    return fused, novelty.reshape(B)


# speedup vs baseline: 1.0477x; 1.0410x over previous
import functools

import jax
import jax.numpy as jnp
from jax import lax
from jax.experimental import pallas as pl
from jax.experimental.pallas import tpu as pltpu
from jax.experimental.pallas import tpu_sc as plsc

B = 128
K = 32768
D = 1024
TOPK = 8
KB = 4096
NUM_KB = K // KB

_NEG_INF = float("-inf")

_SORT8 = [(0, 1), (2, 3), (4, 5), (6, 7), (0, 2), (1, 3), (4, 6), (5, 7),
          (1, 2), (5, 6), (0, 4), (1, 5), (2, 6), (3, 7), (2, 4), (3, 5),
          (1, 2), (3, 4), (5, 6)]
_BITONIC8 = [(0, 4), (1, 5), (2, 6), (3, 7), (0, 2), (1, 3), (4, 6), (5, 7),
             (0, 1), (2, 3), (4, 5), (6, 7)]


def _ce(va, ia, vb, ib):
    m = va >= vb
    return (jnp.where(m, va, vb), jnp.where(m, ia, ib),
            jnp.where(m, vb, va), jnp.where(m, ib, ia))


def _apply_net(net, vs, js):
    vs = list(vs)
    js = list(js)
    for a, b in net:
        vs[a], js[a], vs[b], js[b] = _ce(vs[a], js[a], vs[b], js[b])
    return vs, js


def _top8_of_union(rv, ri, dv, di):
    cs_v = []
    cs_i = []
    for i in range(8):
        m = rv[i] >= dv[7 - i]
        cs_v.append(jnp.where(m, rv[i], dv[7 - i]))
        cs_i.append(jnp.where(m, ri[i], di[7 - i]))
    return _apply_net(_BITONIC8, cs_v, cs_i)


def _qproj_body(qh_ref, wq_ref, bq_ref, qnt_ref):
    qp_t = lax.dot_general(
        wq_ref[...].astype(jnp.bfloat16), qh_ref[...].astype(jnp.bfloat16),
        dimension_numbers=(((1,), (1,)), ((), ())),
        preferred_element_type=jnp.float32)
    qp_t = qp_t + bq_ref[...]
    n2 = jnp.sum(qp_t * qp_t, axis=0, keepdims=True)
    qnt_ref[...] = qp_t / jnp.maximum(jnp.sqrt(n2), 1e-12)


def _qproj(query_hidden, W_q, b_q):
    return pl.pallas_call(
        _qproj_body,
        out_shape=jax.ShapeDtypeStruct((D, B), jnp.float32),
    )(query_hidden, W_q, b_q.reshape(D, 1))


def _merge_block(st, idx, cv_ref, ci_ref):
    g = KB // 64
    sv = st.reshape(g, 8, 8, B)
    si = idx.reshape(g, 8, 8, B)
    vs = [sv[:, c] for c in range(8)]
    js = [si[:, c] for c in range(8)]
    vs, js = _apply_net(_SORT8, vs, js)
    while g > 1:
        h = g // 2
        vs, js = _top8_of_union([v[:h] for v in vs], [x[:h] for x in js],
                                [v[h:] for v in vs], [x[h:] for x in js])
        g = h
    vs = [v[0] for v in vs]
    js = [x[0] for x in js]
    cv = cv_ref[...]
    ci = ci_ref[...]
    rv = [cv[8 * t:8 * t + 8, :] for t in range(8)]
    ri = [ci[8 * t:8 * t + 8, :] for t in range(8)]
    nv, ni = _top8_of_union(rv, ri, vs, js)
    cv_ref[...] = jnp.concatenate(nv, axis=0)
    ci_ref[...] = jnp.concatenate(ni, axis=0)


def _simstopk_body(qnt_ref, keys_ref, tv_ref, ti_ref, cv_ref, ci_ref):
    j = pl.program_id(0)

    @pl.when(j == 0)
    def _init():
        cv_ref[...] = jnp.full((64, B), _NEG_INF, jnp.float32)
        ci_ref[...] = jnp.zeros((64, B), jnp.int32)

    k = keys_ref[...]
    n2 = jnp.sum(k * k, axis=1, keepdims=True)
    inv = 1.0 / jnp.maximum(jnp.sqrt(n2), 1e-12)
    kn = k * inv
    st = lax.dot_general(
        kn.astype(jnp.bfloat16), qnt_ref[...].astype(jnp.bfloat16),
        dimension_numbers=(((1,), (0,)), ((), ())),
        preferred_element_type=jnp.float32)
    idx = lax.broadcasted_iota(jnp.int32, (KB, B), 0) + j * KB
    _merge_block(st, idx, cv_ref, ci_ref)

    @pl.when(j == NUM_KB - 1)
    def _finalize():
        cand_v = cv_ref[...]
        cand_i = ci_ref[...]
        rows_v = []
        rows_i = []
        for _ in range(TOPK):
            m = jnp.max(cand_v, axis=0, keepdims=True)
            eq = cand_v == m
            pick = jnp.min(jnp.where(eq, cand_i, jnp.int32(2**31 - 1)),
                           axis=0, keepdims=True)
            rows_v.append(m)
            rows_i.append(pick)
            cand_v = jnp.where(eq, _NEG_INF, cand_v)
        tv_ref[...] = jnp.concatenate(rows_v, axis=0)
        ti_ref[...] = jnp.concatenate(rows_i, axis=0)


def _simstopk(qn_t, keys):
    return pl.pallas_call(
        _simstopk_body,
        grid=(NUM_KB,),
        in_specs=[
            pl.BlockSpec((D, B), lambda j: (0, 0)),
            pl.BlockSpec((KB, D), lambda j: (j, 0)),
        ],
        out_specs=[
            pl.BlockSpec((TOPK, B), lambda j: (0, 0)),
            pl.BlockSpec((TOPK, B), lambda j: (0, 0)),
        ],
        out_shape=[
            jax.ShapeDtypeStruct((TOPK, B), jnp.float32),
            jax.ShapeDtypeStruct((TOPK, B), jnp.int32),
        ],
        scratch_shapes=[
            pltpu.VMEM((64, B), jnp.float32),
            pltpu.VMEM((64, B), jnp.int32),
        ],
    )(qn_t, keys)


_NW = 32
_ROWS_PER_W = (B * TOPK) // _NW


def _sc_gather(values, idx_flat):
    mesh = plsc.VectorSubcoreMesh(core_axis_name="c", subcore_axis_name="s")

    @functools.partial(
        pl.kernel,
        mesh=mesh,
        out_type=jax.ShapeDtypeStruct((B * TOPK, D), jnp.float32),
        scratch_types=[
            pltpu.VMEM((_ROWS_PER_W,), jnp.int32),
            pltpu.VMEM((_ROWS_PER_W, D), jnp.float32),
            pltpu.SemaphoreType.DMA,
        ],
    )
    def _gather_kernel(values_hbm, idx_hbm, out_hbm, idx_v, rows_v, sem):
        wid = lax.axis_index("s") * 2 + lax.axis_index("c")
        base = wid * _ROWS_PER_W
        pltpu.sync_copy(idx_hbm.at[pl.ds(base, _ROWS_PER_W)], idx_v)
        pltpu.async_copy(values_hbm.at[idx_v], rows_v, sem).wait()
        pltpu.sync_copy(rows_v, out_hbm.at[pl.ds(base, _ROWS_PER_W)])

    return _gather_kernel(values, idx_flat)


def _final_body(g_ref, tv_ref, tok_ref, wv_ref, bv_ref, mg_ref,
                fused_ref, nov_ref):
    tvt = jnp.transpose(tv_ref[...], (1, 0))
    m = jnp.max(tvt, axis=1, keepdims=True)
    e = jnp.exp(tvt - m)
    w = e / jnp.sum(e, axis=1, keepdims=True)
    nov_ref[...] = 1.0 - m

    r = jnp.zeros((B, D), jnp.float32)
    for i in range(TOPK):
        r = r + g_ref[pl.ds(i * B, B), :] * w[:, i:i + 1]

    mh = lax.dot_general(
        r.astype(jnp.bfloat16), wv_ref[...].astype(jnp.bfloat16),
        dimension_numbers=(((1,), (1,)), ((), ())),
        preferred_element_type=jnp.float32)
    mh = mh + bv_ref[...]
    gate = jax.nn.sigmoid(mg_ref[0, 0])
    fused_ref[...] = (1.0 - gate) * tok_ref[...] + gate * mh


def _final(gathered, top_vals, token_embed, W_v, b_v, memory_gate):
    return pl.pallas_call(
        _final_body,
        out_shape=[
            jax.ShapeDtypeStruct((B, D), jnp.float32),
            jax.ShapeDtypeStruct((B, 1), jnp.float32),
        ],
    )(gathered, top_vals, token_embed, W_v, b_v.reshape(1, D),
      memory_gate.reshape(1, 1))


def kernel(query_hidden, keys, values, token_embed, W_q, b_q, W_v, b_v,
           memory_gate):
    qn_t = _qproj(query_hidden, W_q, b_q)
    top_vals, top_idx = _simstopk(qn_t, keys)
    gathered = _sc_gather(values, top_idx.reshape(B * TOPK))
    fused, novelty = _final(gathered, top_vals, token_embed, W_v, b_v,
                            memory_gate)
    return fused, novelty.reshape(B)


# speedup vs baseline: 1.0568x; 1.0087x over previous
import functools

import jax
import jax.numpy as jnp
from jax import lax
from jax.experimental import pallas as pl
from jax.experimental.pallas import tpu as pltpu
from jax.experimental.pallas import tpu_sc as plsc

B = 128
K = 32768
D = 1024
TOPK = 8
KB = 4096
NUM_KB = K // KB

_NEG_INF = float("-inf")

_SORT8 = [(0, 1), (2, 3), (4, 5), (6, 7), (0, 2), (1, 3), (4, 6), (5, 7),
          (1, 2), (5, 6), (0, 4), (1, 5), (2, 6), (3, 7), (2, 4), (3, 5),
          (1, 2), (3, 4), (5, 6)]
_BITONIC8 = [(0, 4), (1, 5), (2, 6), (3, 7), (0, 2), (1, 3), (4, 6), (5, 7),
             (0, 1), (2, 3), (4, 5), (6, 7)]


def _ce(va, ia, vb, ib):
    m = va >= vb
    return (jnp.where(m, va, vb), jnp.where(m, ia, ib),
            jnp.where(m, vb, va), jnp.where(m, ib, ia))


def _apply_net(net, vs, js):
    vs = list(vs)
    js = list(js)
    for a, b in net:
        vs[a], js[a], vs[b], js[b] = _ce(vs[a], js[a], vs[b], js[b])
    return vs, js


def _top8_of_union(rv, ri, dv, di):
    cs_v = []
    cs_i = []
    for i in range(8):
        m = rv[i] >= dv[7 - i]
        cs_v.append(jnp.where(m, rv[i], dv[7 - i]))
        cs_i.append(jnp.where(m, ri[i], di[7 - i]))
    return _apply_net(_BITONIC8, cs_v, cs_i)


def _qproj_body(qh_ref, wq_ref, bq_ref, qnt_ref):
    qp_t = lax.dot_general(
        wq_ref[...].astype(jnp.bfloat16), qh_ref[...].astype(jnp.bfloat16),
        dimension_numbers=(((1,), (1,)), ((), ())),
        preferred_element_type=jnp.float32)
    qp_t = qp_t + bq_ref[...]
    n2 = jnp.sum(qp_t * qp_t, axis=0, keepdims=True)
    qnt_ref[...] = qp_t / jnp.maximum(jnp.sqrt(n2), 1e-12)


def _qproj(query_hidden, W_q, b_q):
    return pl.pallas_call(
        _qproj_body,
        out_shape=jax.ShapeDtypeStruct((D, B), jnp.float32),
    )(query_hidden, W_q, b_q.reshape(D, 1))


def _merge_block(st, idx, cv_ref, ci_ref):
    g = KB // 64
    sv = st.reshape(g, 8, 8, B)
    si = idx.reshape(g, 8, 8, B)
    vs = [sv[:, c] for c in range(8)]
    js = [si[:, c] for c in range(8)]
    vs, js = _apply_net(_SORT8, vs, js)
    while g > 1:
        h = g // 2
        vs, js = _top8_of_union([v[:h] for v in vs], [x[:h] for x in js],
                                [v[h:] for v in vs], [x[h:] for x in js])
        g = h
    vs = [v[0] for v in vs]
    js = [x[0] for x in js]
    cv = cv_ref[...]
    ci = ci_ref[...]
    rv = [cv[8 * t:8 * t + 8, :] for t in range(8)]
    ri = [ci[8 * t:8 * t + 8, :] for t in range(8)]
    nv, ni = _top8_of_union(rv, ri, vs, js)
    cv_ref[...] = jnp.concatenate(nv, axis=0)
    ci_ref[...] = jnp.concatenate(ni, axis=0)


def _simstopk_body(qnt_ref, keys_ref, tv_ref, ti_ref, cv_ref, ci_ref):
    j = pl.program_id(0)

    @pl.when(j == 0)
    def _init():
        cv_ref[...] = jnp.full((64, B), _NEG_INF, jnp.float32)
        ci_ref[...] = jnp.zeros((64, B), jnp.int32)

    k = keys_ref[...]
    n2 = jnp.sum(k * k, axis=1, keepdims=True)
    inv = 1.0 / jnp.maximum(jnp.sqrt(n2), 1e-12)
    kn = k * inv
    st = lax.dot_general(
        kn, qnt_ref[...],
        dimension_numbers=(((1,), (0,)), ((), ())),
        precision=lax.Precision.DEFAULT,
        preferred_element_type=jnp.float32)
    idx = lax.broadcasted_iota(jnp.int32, (KB, B), 0) + j * KB
    _merge_block(st, idx, cv_ref, ci_ref)

    @pl.when(j == NUM_KB - 1)
    def _finalize():
        cand_v = cv_ref[...]
        cand_i = ci_ref[...]
        rows_v = []
        rows_i = []
        for _ in range(TOPK):
            m = jnp.max(cand_v, axis=0, keepdims=True)
            eq = cand_v == m
            pick = jnp.min(jnp.where(eq, cand_i, jnp.int32(2**31 - 1)),
                           axis=0, keepdims=True)
            rows_v.append(m)
            rows_i.append(pick)
            cand_v = jnp.where(eq, _NEG_INF, cand_v)
        tv_ref[...] = jnp.concatenate(rows_v, axis=0)
        ti_ref[...] = jnp.concatenate(rows_i, axis=0)


def _simstopk(qn_t, keys):
    return pl.pallas_call(
        _simstopk_body,
        grid=(NUM_KB,),
        in_specs=[
            pl.BlockSpec((D, B), lambda j: (0, 0)),
            pl.BlockSpec((KB, D), lambda j: (j, 0)),
        ],
        out_specs=[
            pl.BlockSpec((TOPK, B), lambda j: (0, 0)),
            pl.BlockSpec((TOPK, B), lambda j: (0, 0)),
        ],
        out_shape=[
            jax.ShapeDtypeStruct((TOPK, B), jnp.float32),
            jax.ShapeDtypeStruct((TOPK, B), jnp.int32),
        ],
        scratch_shapes=[
            pltpu.VMEM((64, B), jnp.float32),
            pltpu.VMEM((64, B), jnp.int32),
        ],
    )(qn_t, keys)


_NW = 32
_ROWS_PER_W = (B * TOPK) // _NW


def _sc_gather(values, idx_flat):
    mesh = plsc.VectorSubcoreMesh(core_axis_name="c", subcore_axis_name="s")

    @functools.partial(
        pl.kernel,
        mesh=mesh,
        out_type=jax.ShapeDtypeStruct((B * TOPK, D), jnp.float32),
        scratch_types=[
            pltpu.VMEM((_ROWS_PER_W,), jnp.int32),
            pltpu.VMEM((_ROWS_PER_W, D), jnp.float32),
            pltpu.SemaphoreType.DMA,
        ],
    )
    def _gather_kernel(values_hbm, idx_hbm, out_hbm, idx_v, rows_v, sem):
        wid = lax.axis_index("s") * 2 + lax.axis_index("c")
        base = wid * _ROWS_PER_W
        pltpu.sync_copy(idx_hbm.at[pl.ds(base, _ROWS_PER_W)], idx_v)
        pltpu.async_copy(values_hbm.at[idx_v], rows_v, sem).wait()
        pltpu.sync_copy(rows_v, out_hbm.at[pl.ds(base, _ROWS_PER_W)])

    return _gather_kernel(values, idx_flat)


def _final_body(g_ref, tv_ref, tok_ref, wv_ref, bv_ref, mg_ref,
                fused_ref, nov_ref):
    tvt = jnp.transpose(tv_ref[...], (1, 0))
    m = jnp.max(tvt, axis=1, keepdims=True)
    e = jnp.exp(tvt - m)
    w = e / jnp.sum(e, axis=1, keepdims=True)
    nov_ref[...] = 1.0 - m

    r = jnp.zeros((B, D), jnp.float32)
    for i in range(TOPK):
        r = r + g_ref[pl.ds(i * B, B), :] * w[:, i:i + 1]

    mh = lax.dot_general(
        r.astype(jnp.bfloat16), wv_ref[...].astype(jnp.bfloat16),
        dimension_numbers=(((1,), (1,)), ((), ())),
        preferred_element_type=jnp.float32)
    mh = mh + bv_ref[...]
    gate = jax.nn.sigmoid(mg_ref[0, 0])
    fused_ref[...] = (1.0 - gate) * tok_ref[...] + gate * mh


def _final(gathered, top_vals, token_embed, W_v, b_v, memory_gate):
    return pl.pallas_call(
        _final_body,
        out_shape=[
            jax.ShapeDtypeStruct((B, D), jnp.float32),
            jax.ShapeDtypeStruct((B, 1), jnp.float32),
        ],
    )(gathered, top_vals, token_embed, W_v, b_v.reshape(1, D),
      memory_gate.reshape(1, 1))


def kernel(query_hidden, keys, values, token_embed, W_q, b_q, W_v, b_v,
           memory_gate):
    qn_t = _qproj(query_hidden, W_q, b_q)
    top_vals, top_idx = _simstopk(qn_t, keys)
    gathered = _sc_gather(values, top_idx.reshape(B * TOPK))
    fused, novelty = _final(gathered, top_vals, token_embed, W_v, b_v,
                            memory_gate)
    return fused, novelty.reshape(B)
